# baseline (speedup 1.0000x reference)
import math
from functools import lru_cache, partial

import numpy as np
import jax
import jax.numpy as jnp
from jax.experimental import pallas as pl
from jax.experimental.pallas import tpu as pltpu

_VMEM_LIMIT_BYTES = 48 * 1024 * 1024


def _nearest_indices(in_dim: int, out_dim: int) -> np.ndarray:
    src = np.floor(np.arange(out_dim, dtype=np.float32) * np.float32(in_dim / out_dim))
    return np.clip(src.astype(np.int64), 0, in_dim - 1)


@lru_cache(maxsize=16)
def _sel_w_mat(w_in: int, w_out: int):
    idx = _nearest_indices(w_in, w_out)
    m = np.zeros((w_in, w_out), dtype=np.float32)
    m[idx, np.arange(w_out)] = 1.0
    return jnp.asarray(m)


def _upsample_kernel(sel_w_ref, x_ref, o_ref, *, sf_h):
    c, h_in, w_in = x_ref.shape[1], x_ref.shape[2], x_ref.shape[3]
    w_out = o_ref.shape[3]
    x2d = x_ref[0].reshape(c * h_in, w_in)
    t = jnp.dot(x2d, sel_w_ref[...], preferred_element_type=jnp.float32)
    t3 = t.reshape(c, h_in, w_out)
    for j in range(sf_h):
        o_ref[0, :, j::sf_h, :] = t3


def kernel(x):
    N, C, H_in, W_in = x.shape
    sf_h = sf_w = 2
    H_out, W_out = H_in * sf_h, W_in * sf_w

    orig_dtype = x.dtype
    if not jnp.issubdtype(x.dtype, jnp.floating):
        x = x.astype(jnp.float32)

    sel_w = _sel_w_mat(W_in, W_out).astype(x.dtype)

    out = pl.pallas_call(
        partial(_upsample_kernel, sf_h=sf_h),
        out_shape=jax.ShapeDtypeStruct((N, C, H_out, W_out), x.dtype),
        grid=(N,),
        in_specs=[
            pl.BlockSpec((W_in, W_out), lambda n: (0, 0)),
            pl.BlockSpec((1, C, H_in, W_in), lambda n: (n, 0, 0, 0)),
        ],
        out_specs=pl.BlockSpec((1, C, H_out, W_out), lambda n: (n, 0, 0, 0)),
        compiler_params=pltpu.CompilerParams(
            dimension_semantics=("parallel",),
            vmem_limit_bytes=_VMEM_LIMIT_BYTES,
        ),
    )(sel_w, x)

    if out.dtype != orig_dtype:
        out = out.astype(orig_dtype)
    return out
